# baseline (speedup 1.0000x reference)
import functools

import jax
import jax.numpy as jnp
from jax import lax
from jax.experimental import pallas as pl
from jax.experimental.pallas import tpu as pltpu
from jax.experimental.pallas import tpu_sc as plsc

B = 4096
D = 32

_info = plsc.get_sparse_core_info()
_NC, _NS = _info.num_cores, _info.num_subcores
_NW = _NC * _NS
_BPW = B // _NW

_mesh = plsc.VectorSubcoreMesh(core_axis_name="c", subcore_axis_name="s")


@functools.partial(
    pl.kernel,
    mesh=_mesh,
    out_type=[
        jax.ShapeDtypeStruct((B, D), jnp.float32),
        jax.ShapeDtypeStruct((B, D), jnp.float32),
    ],
    scratch_types=[
        pltpu.VMEM((_BPW,), jnp.int32),
        pltpu.VMEM((_BPW, D), jnp.float32),
        pltpu.VMEM((_BPW,), jnp.int32),
        pltpu.VMEM((_BPW, D), jnp.float32),
        pltpu.SemaphoreType.DMA,
        pltpu.SemaphoreType.DMA,
    ],
    compiler_params=pltpu.CompilerParams(use_tc_tiling_on_sc=False),
)
def _sc_gather(u_hbm, i_hbm, ut_hbm, it_hbm, w_hbm, h_hbm,
               uidx_v, urows_v, iidx_v, irows_v, sem_u, sem_i):
    wid = lax.axis_index("s") * _NC + lax.axis_index("c")
    base = wid * _BPW
    pltpu.sync_copy(u_hbm.at[pl.ds(base, _BPW)], uidx_v)
    pltpu.sync_copy(i_hbm.at[pl.ds(base, _BPW)], iidx_v)
    cp_u = pltpu.async_copy(ut_hbm.at[uidx_v], urows_v, sem_u)
    cp_i = pltpu.async_copy(it_hbm.at[iidx_v], irows_v, sem_i)
    cp_u.wait()
    cp_i.wait()
    pltpu.sync_copy(urows_v, w_hbm.at[pl.ds(base, _BPW)])
    pltpu.sync_copy(irows_v, h_hbm.at[pl.ds(base, _BPW)])


_BM = 512


def _mm_body(h_ref, w_ref, out_ref):
    out_ref[...] = lax.dot_general(
        h_ref[...], w_ref[...],
        (((1,), (1,)), ((), ())),
        preferred_element_type=jnp.float32,
    )


@jax.jit
def _tc_matmul(h, w):
    return pl.pallas_call(
        _mm_body,
        grid=(B // _BM,),
        in_specs=[
            pl.BlockSpec((_BM, D), lambda m: (m, 0)),
            pl.BlockSpec((B, D), lambda m: (0, 0)),
        ],
        out_specs=pl.BlockSpec((_BM, B), lambda m: (m, 0)),
        out_shape=jax.ShapeDtypeStruct((B, B), jnp.float32),
    )(h, w)


@jax.jit
def kernel(u, i, user_table, item_table):
    w, h = _sc_gather(u, i, user_table, item_table)
    return _tc_matmul(h, w)
